# baseline (speedup 1.0000x reference)
import functools

import jax
import jax.numpy as jnp
from jax import lax
from jax.experimental import pallas as pl
from jax.experimental.pallas import tpu as pltpu
from jax.experimental.pallas import tpu_sc as plsc

_BATCH = 4
_ROWS = 8192
_DIM = 1024
_NC = 2
_NS = 16
_NW = _NC * _NS
_ROWS_PER_W = _ROWS // _NW
_CHUNK = 64
_STEPS = _ROWS_PER_W // _CHUNK

_mesh = plsc.VectorSubcoreMesh(core_axis_name="c", subcore_axis_name="s")


@functools.partial(
    pl.kernel,
    mesh=_mesh,
    out_type=jax.ShapeDtypeStruct((_BATCH * _ROWS, _DIM), jnp.float32),
    scratch_types=[
        pltpu.VMEM((_CHUNK, _DIM), jnp.float32),
        pltpu.SemaphoreType.DMA,
    ],
)
def _broadcast_table(table_hbm, out_hbm, buf, wsem):
    wid = lax.axis_index("s") * _NC + lax.axis_index("c")
    base = wid * _ROWS_PER_W
    for s in range(_STEPS):
        r = base + s * _CHUNK
        pltpu.sync_copy(table_hbm.at[pl.ds(r, _CHUNK)], buf)
        writes = [
            pltpu.async_copy(
                buf, out_hbm.at[pl.ds(b * _ROWS + r, _CHUNK)], wsem)
            for b in range(_BATCH)
        ]
        for w in writes:
            w.wait()


def kernel(x, table):
    del x
    flat = _broadcast_table(table)
    return flat.reshape(_BATCH, _ROWS, _DIM)


# speedup vs baseline: 1.0335x; 1.0335x over previous
import functools

import jax
import jax.numpy as jnp
from jax import lax
from jax.experimental import pallas as pl
from jax.experimental.pallas import tpu as pltpu
from jax.experimental.pallas import tpu_sc as plsc

_BATCH = 4
_ROWS = 8192
_DIM = 1024
_NC = 2
_NS = 16
_NW = _NC * _NS
_ROWS_PER_W = _ROWS // _NW
_CHUNKS = (96, 96, 64)
_CHUNK_MAX = max(_CHUNKS)

_mesh = plsc.VectorSubcoreMesh(core_axis_name="c", subcore_axis_name="s")


@functools.partial(
    pl.kernel,
    mesh=_mesh,
    out_type=jax.ShapeDtypeStruct((_BATCH * _ROWS, _DIM), jnp.float32),
    scratch_types=[pltpu.VMEM((_CHUNK_MAX, _DIM), jnp.float32)],
)
def _broadcast_table(table_hbm, out_hbm, buf):
    wid = lax.axis_index("s") * _NC + lax.axis_index("c")
    base = wid * _ROWS_PER_W
    off = 0
    for chunk in _CHUNKS:
        r = base + off
        pltpu.sync_copy(table_hbm.at[pl.ds(r, chunk)], buf.at[pl.ds(0, chunk)])
        for b in range(_BATCH):
            pltpu.sync_copy(
                buf.at[pl.ds(0, chunk)],
                out_hbm.at[pl.ds(b * _ROWS + r, chunk)])
        off += chunk


def kernel(x, table):
    del x
    flat = _broadcast_table(table)
    return flat.reshape(_BATCH, _ROWS, _DIM)


# speedup vs baseline: 1.0351x; 1.0016x over previous
import functools

import jax
import jax.numpy as jnp
from jax import lax
from jax.experimental import pallas as pl
from jax.experimental.pallas import tpu as pltpu
from jax.experimental.pallas import tpu_sc as plsc

_BATCH = 4
_ROWS = 8192
_DIM = 1024
_NC = 2
_NS = 16
_NW = _NC * _NS
_ROWS_PER_W = _ROWS // _NW
_CHUNKS = (120, 120, 16)
_CHUNK_MAX = max(_CHUNKS)

_mesh = plsc.VectorSubcoreMesh(core_axis_name="c", subcore_axis_name="s")


@functools.partial(
    pl.kernel,
    mesh=_mesh,
    out_type=jax.ShapeDtypeStruct((_BATCH * _ROWS, _DIM), jnp.float32),
    scratch_types=[pltpu.VMEM((_CHUNK_MAX, _DIM), jnp.float32)],
)
def _broadcast_table(table_hbm, out_hbm, buf):
    wid = lax.axis_index("s") * _NC + lax.axis_index("c")
    base = wid * _ROWS_PER_W
    off = 0
    for chunk in _CHUNKS:
        r = base + off
        pltpu.sync_copy(table_hbm.at[pl.ds(r, chunk)], buf.at[pl.ds(0, chunk)])
        for b in range(_BATCH):
            pltpu.sync_copy(
                buf.at[pl.ds(0, chunk)],
                out_hbm.at[pl.ds(b * _ROWS + r, chunk)])
        off += chunk


def kernel(x, table):
    del x
    flat = _broadcast_table(table)
    return flat.reshape(_BATCH, _ROWS, _DIM)
